# baseline (speedup 1.0000x reference)
import functools
import math

import jax
import jax.numpy as jnp
import numpy as np
from jax import lax
from jax.experimental import pallas as pl
from jax.experimental.pallas import tpu as pltpu
from jax.experimental.pallas import tpu_sc as plsc

_THRESH = 0.7
_TB_HARD = int(np.float32(-np.log(np.float32(_THRESH))).view(np.int32)) + 1

_NC = 2
_NS = 16
_NW = _NC * _NS
_LANES = 16
_PACK = 32


def _ce_loss_body(logits_ref, labels_ref, loss_ref):
    x = logits_ref[0]
    lbl = labels_ref[0]
    m = jnp.max(x, axis=0)
    s = jnp.sum(jnp.exp(x - m[None, :, :]), axis=0)
    cidx = lax.broadcasted_iota(jnp.int32, x.shape, 0)
    pick = jnp.sum(jnp.where(cidx == lbl[None, :, :], x, 0.0), axis=0)
    raw = jnp.log(s) + m - pick
    loss_ref[...] = jnp.clip(raw, 0.0, jnp.float32(np.finfo(np.float32).max))


def _ce_loss(logits, labels, hb, q, bq):
    b, c, h, w = logits.shape
    nj = h // hb
    return pl.pallas_call(
        _ce_loss_body,
        grid=(bq, nj),
        in_specs=[
            pl.BlockSpec((1, c, hb, w), lambda i, j: (q * bq + i, 0, j, 0)),
            pl.BlockSpec((1, hb, w), lambda i, j: (q * bq + i, j, 0)),
        ],
        out_specs=pl.BlockSpec((hb, w), lambda i, j: (i * nj + j, 0)),
        out_shape=jax.ShapeDtypeStruct((bq * h, w), jnp.float32),
    )(logits, labels)


def _make_sc_count_sum(rows, w, chunk_rows):
    per_w_rows = rows // _NW
    n_chunks = per_w_rows // chunk_rows
    mesh = plsc.VectorSubcoreMesh(core_axis_name="c", subcore_axis_name="s")

    @functools.partial(
        pl.kernel,
        mesh=mesh,
        out_type=[
            jax.ShapeDtypeStruct((_NW * _LANES,), jnp.float32),
            jax.ShapeDtypeStruct((_NW * _LANES,), jnp.int32),
        ],
        scratch_types=[
            pltpu.VMEM((chunk_rows, w), jnp.float32),
            pltpu.VMEM((chunk_rows, w), jnp.float32),
            pltpu.VMEM((_LANES,), jnp.float32),
            pltpu.VMEM((_LANES,), jnp.float32),
            pltpu.VMEM((_LANES,), jnp.int32),
            pltpu.SemaphoreType.DMA,
            pltpu.SemaphoreType.DMA,
        ],
    )
    def sc_count_sum(loss_hbm, t_hbm, sum_out, cnt_out,
                     buf0, buf1, tbuf, osbuf, ocbuf, sem0, sem1):
        wid = lax.axis_index("s") * _NC + lax.axis_index("c")
        base = wid * per_w_rows
        pltpu.sync_copy(t_hbm, tbuf)
        tvec = tbuf[...]
        bufs = (buf0, buf1)
        sems = (sem0, sem1)
        handles = [None, None]
        handles[0] = pltpu.async_copy(
            loss_hbm.at[pl.ds(base, chunk_rows)], buf0, sem0)
        s_acc = jnp.zeros((_LANES,), jnp.float32)
        c_acc = jnp.zeros((_LANES,), jnp.int32)
        for j in range(n_chunks):
            if j + 1 < n_chunks:
                handles[(j + 1) % 2] = pltpu.async_copy(
                    loss_hbm.at[pl.ds(base + (j + 1) * chunk_rows, chunk_rows)],
                    bufs[(j + 1) % 2], sems[(j + 1) % 2])
            handles[j % 2].wait()
            buf = bufs[j % 2]

            def body(i, carry, buf=buf):
                s, cnt = carry
                r = i // 8
                cb = (i % 8) * (4 * _LANES)
                for u in range(4):
                    v = buf[r, pl.ds(cb + u * _LANES, _LANES)]
                    msk = v >= tvec
                    s = s + jnp.where(msk, v, 0.0)
                    cnt = cnt + jnp.where(msk, 1, 0)
                return (s, cnt)

            s_acc, c_acc = lax.fori_loop(
                0, chunk_rows * w // (4 * _LANES), body, (s_acc, c_acc))
        osbuf[...] = s_acc
        ocbuf[...] = c_acc
        pltpu.sync_copy(osbuf, sum_out.at[pl.ds(wid * _LANES, _LANES)])
        pltpu.sync_copy(ocbuf, cnt_out.at[pl.ds(wid * _LANES, _LANES)])

    return sc_count_sum


def _count_sum(sc_kernel, loss_parts, t_bits_scalar):
    tf = lax.bitcast_convert_type(jnp.int32(t_bits_scalar), jnp.float32)
    t = jnp.full((_LANES,), tf, dtype=jnp.float32)
    s_tot = jnp.float32(0.0)
    c_tot = jnp.int32(0)
    for part in loss_parts:
        sums, cnts = sc_kernel(part, t)
        s_tot = s_tot + jnp.sum(sums)
        c_tot = c_tot + jnp.sum(cnts)
    return s_tot, c_tot


def kernel(logits, labels):
    b, c, h, w = logits.shape
    n = b * h * w
    n_min = int(n * 0.25)

    nq = 4
    bq = b // nq
    parts = [_ce_loss(logits, labels, 512, q, bq) for q in range(nq)]

    sc_kernel = _make_sc_count_sum(bq * h, w, chunk_rows=32)

    s_hard, n_hard = _count_sum(sc_kernel, parts, _TB_HARD)
    mean_hard = s_hard / n_hard.astype(jnp.float32)

    def topk_mean(_):
        def cond_fn(state):
            lo, hi = state
            return hi - lo > 1

        def body_fn(state):
            lo, hi = state
            mid = lo + (hi - lo) // 2
            _, cnt = _count_sum(sc_kernel, parts, mid)
            ge = cnt >= n_min
            return (jnp.where(ge, mid, lo), jnp.where(ge, hi, mid))

        lo0 = jnp.int32(0)
        hi0 = jnp.int32(0x7F800000)
        v_bits, _ = lax.while_loop(cond_fn, body_fn, (lo0, hi0))
        v = lax.bitcast_convert_type(v_bits, jnp.float32)
        s_gt, c_gt = _count_sum(sc_kernel, parts, v_bits + 1)
        topk_sum = s_gt + (n_min - c_gt).astype(jnp.float32) * v
        return topk_sum / jnp.float32(n_min)

    return lax.cond(n_hard < n_min, topk_mean,
                    lambda _: mean_hard, operand=None)


# speedup vs baseline: 1.1748x; 1.1748x over previous
import functools
import math

import jax
import jax.numpy as jnp
import numpy as np
from jax import lax
from jax.experimental import pallas as pl
from jax.experimental.pallas import tpu as pltpu
from jax.experimental.pallas import tpu_sc as plsc

_THRESH = 0.7
_TB_HARD = int(np.float32(-np.log(np.float32(_THRESH))).view(np.int32)) + 1

_NC = 2
_NS = 16
_NW = _NC * _NS
_LANES = 16
_PACK = 32


def _ce_loss_body(logits_ref, labels_ref, loss_ref):
    x = logits_ref[0]
    lbl = labels_ref[0]
    m = jnp.max(x, axis=0)
    s = jnp.sum(jnp.exp(x - m[None, :, :]), axis=0)
    cidx = lax.broadcasted_iota(jnp.int32, x.shape, 0)
    pick = jnp.sum(jnp.where(cidx == lbl[None, :, :], x, 0.0), axis=0)
    raw = jnp.log(s) + m - pick
    loss_ref[...] = jnp.clip(raw, 0.0, jnp.float32(np.finfo(np.float32).max))


def _ce_loss(logits, labels, hb, q, bq):
    b, c, h, w = logits.shape
    nj = h // hb
    return pl.pallas_call(
        _ce_loss_body,
        grid=(bq, nj),
        in_specs=[
            pl.BlockSpec((1, c, hb, w), lambda i, j: (q * bq + i, 0, j, 0)),
            pl.BlockSpec((1, hb, w), lambda i, j: (q * bq + i, j, 0)),
        ],
        out_specs=pl.BlockSpec((hb, w), lambda i, j: (i * nj + j, 0)),
        out_shape=jax.ShapeDtypeStruct((bq * h, w), jnp.float32),
    )(logits, labels)


def _make_sc_count_sum(rows, w, chunk_rows):
    per_w_rows = rows // _NW
    n_chunks = per_w_rows // chunk_rows
    mesh = plsc.VectorSubcoreMesh(core_axis_name="c", subcore_axis_name="s")

    @functools.partial(
        pl.kernel,
        mesh=mesh,
        out_type=[
            jax.ShapeDtypeStruct((_NW * _LANES,), jnp.float32),
            jax.ShapeDtypeStruct((_NW * _LANES,), jnp.int32),
        ],
        scratch_types=[
            pltpu.VMEM((chunk_rows, w), jnp.float32),
            pltpu.VMEM((chunk_rows, w), jnp.float32),
            pltpu.VMEM((_LANES,), jnp.float32),
            pltpu.VMEM((_LANES,), jnp.float32),
            pltpu.VMEM((_LANES,), jnp.int32),
            pltpu.SemaphoreType.DMA,
            pltpu.SemaphoreType.DMA,
        ],
    )
    def sc_count_sum(loss_hbm, t_hbm, sum_out, cnt_out,
                     buf0, buf1, tbuf, osbuf, ocbuf, sem0, sem1):
        wid = lax.axis_index("s") * _NC + lax.axis_index("c")
        base = wid * per_w_rows
        pltpu.sync_copy(t_hbm, tbuf)
        tvec = tbuf[...]
        bufs = (buf0, buf1)
        sems = (sem0, sem1)
        handles = [None, None]
        handles[0] = pltpu.async_copy(
            loss_hbm.at[pl.ds(base, chunk_rows)], buf0, sem0)
        s_acc = jnp.zeros((_LANES,), jnp.float32)
        c_acc = jnp.zeros((_LANES,), jnp.int32)
        for j in range(n_chunks):
            if j + 1 < n_chunks:
                handles[(j + 1) % 2] = pltpu.async_copy(
                    loss_hbm.at[pl.ds(base + (j + 1) * chunk_rows, chunk_rows)],
                    bufs[(j + 1) % 2], sems[(j + 1) % 2])
            handles[j % 2].wait()
            buf = bufs[j % 2]

            def body(i, carry, buf=buf):
                s, cnt = carry
                r = i // 4
                cb = (i % 4) * (8 * _LANES)
                for u in range(8):
                    v = buf[r, pl.ds(cb + u * _LANES, _LANES)]
                    msk = v >= tvec
                    s = s + jnp.where(msk, v, 0.0)
                    cnt = cnt + jnp.where(msk, 1, 0)
                return (s, cnt)

            s_acc, c_acc = lax.fori_loop(
                0, chunk_rows * w // (8 * _LANES), body, (s_acc, c_acc))
        osbuf[...] = s_acc
        ocbuf[...] = c_acc
        pltpu.sync_copy(osbuf, sum_out.at[pl.ds(wid * _LANES, _LANES)])
        pltpu.sync_copy(ocbuf, cnt_out.at[pl.ds(wid * _LANES, _LANES)])

    return sc_count_sum


def _count_sum(sc_kernel, loss_parts, t_bits_scalar):
    tf = lax.bitcast_convert_type(jnp.int32(t_bits_scalar), jnp.float32)
    t = jnp.full((_LANES,), tf, dtype=jnp.float32)
    s_tot = jnp.float32(0.0)
    c_tot = jnp.int32(0)
    for part in loss_parts:
        sums, cnts = sc_kernel(part, t)
        s_tot = s_tot + jnp.sum(sums)
        c_tot = c_tot + jnp.sum(cnts)
    return s_tot, c_tot


def kernel(logits, labels):
    b, c, h, w = logits.shape
    n = b * h * w
    n_min = int(n * 0.25)

    nq = 1
    bq = b // nq
    parts = [_ce_loss(logits, labels, 512, q, bq) for q in range(nq)]

    sc_kernel = _make_sc_count_sum(bq * h, w, chunk_rows=64)

    s_hard, n_hard = _count_sum(sc_kernel, parts, _TB_HARD)
    mean_hard = s_hard / n_hard.astype(jnp.float32)

    def topk_mean(_):
        def cond_fn(state):
            lo, hi = state
            return hi - lo > 1

        def body_fn(state):
            lo, hi = state
            mid = lo + (hi - lo) // 2
            _, cnt = _count_sum(sc_kernel, parts, mid)
            ge = cnt >= n_min
            return (jnp.where(ge, mid, lo), jnp.where(ge, hi, mid))

        lo0 = jnp.int32(0)
        hi0 = jnp.int32(0x7F800000)
        v_bits, _ = lax.while_loop(cond_fn, body_fn, (lo0, hi0))
        v = lax.bitcast_convert_type(v_bits, jnp.float32)
        s_gt, c_gt = _count_sum(sc_kernel, parts, v_bits + 1)
        topk_sum = s_gt + (n_min - c_gt).astype(jnp.float32) * v
        return topk_sum / jnp.float32(n_min)

    return lax.cond(n_hard < n_min, topk_mean,
                    lambda _: mean_hard, operand=None)


# speedup vs baseline: 1.1852x; 1.0088x over previous
import functools
import math

import jax
import jax.numpy as jnp
import numpy as np
from jax import lax
from jax.experimental import pallas as pl
from jax.experimental.pallas import tpu as pltpu
from jax.experimental.pallas import tpu_sc as plsc

_THRESH = 0.7
_TB_HARD = int(np.float32(-np.log(np.float32(_THRESH))).view(np.int32)) + 1

_NC = 2
_NS = 16
_NW = _NC * _NS
_LANES = 16
_PACK = 32


def _ce_loss_body(logits_ref, labels_ref, loss_ref):
    x = logits_ref[0]
    lbl = labels_ref[0]
    m = jnp.max(x, axis=0)
    s = jnp.sum(jnp.exp(x - m[None, :, :]), axis=0)
    cidx = lax.broadcasted_iota(jnp.int32, x.shape, 0)
    pick = jnp.sum(jnp.where(cidx == lbl[None, :, :], x, 0.0), axis=0)
    raw = jnp.log(s) + m - pick
    loss_ref[...] = jnp.clip(raw, 0.0, jnp.float32(np.finfo(np.float32).max))


def _ce_loss(logits, labels, hb, q, bq):
    b, c, h, w = logits.shape
    nj = h // hb
    return pl.pallas_call(
        _ce_loss_body,
        grid=(bq, nj),
        in_specs=[
            pl.BlockSpec((1, c, hb, w), lambda i, j: (q * bq + i, 0, j, 0)),
            pl.BlockSpec((1, hb, w), lambda i, j: (q * bq + i, j, 0)),
        ],
        out_specs=pl.BlockSpec((hb, w), lambda i, j: (i * nj + j, 0)),
        out_shape=jax.ShapeDtypeStruct((bq * h, w), jnp.float32),
    )(logits, labels)


def _make_sc_count_sum(rows, w, chunk_rows):
    per_w_rows = rows // _NW
    n_chunks = per_w_rows // chunk_rows
    mesh = plsc.VectorSubcoreMesh(core_axis_name="c", subcore_axis_name="s")

    @functools.partial(
        pl.kernel,
        mesh=mesh,
        out_type=[
            jax.ShapeDtypeStruct((_NW * _LANES,), jnp.float32),
            jax.ShapeDtypeStruct((_NW * _LANES,), jnp.int32),
        ],
        scratch_types=[
            pltpu.VMEM((chunk_rows, w), jnp.float32),
            pltpu.VMEM((chunk_rows, w), jnp.float32),
            pltpu.VMEM((_LANES,), jnp.float32),
            pltpu.VMEM((_LANES,), jnp.float32),
            pltpu.VMEM((_LANES,), jnp.int32),
            pltpu.SemaphoreType.DMA,
            pltpu.SemaphoreType.DMA,
        ],
    )
    def sc_count_sum(loss_hbm, t_hbm, sum_out, cnt_out,
                     buf0, buf1, tbuf, osbuf, ocbuf, sem0, sem1):
        wid = lax.axis_index("s") * _NC + lax.axis_index("c")
        base = wid * per_w_rows
        pltpu.sync_copy(t_hbm, tbuf)
        tvec = tbuf[...]
        bufs = (buf0, buf1)
        sems = (sem0, sem1)
        handles = [None, None]
        handles[0] = pltpu.async_copy(
            loss_hbm.at[pl.ds(base, chunk_rows)], buf0, sem0)
        s_acc = jnp.zeros((_LANES,), jnp.float32)
        c_acc = jnp.zeros((_LANES,), jnp.int32)
        for j in range(n_chunks):
            if j + 1 < n_chunks:
                handles[(j + 1) % 2] = pltpu.async_copy(
                    loss_hbm.at[pl.ds(base + (j + 1) * chunk_rows, chunk_rows)],
                    bufs[(j + 1) % 2], sems[(j + 1) % 2])
            handles[j % 2].wait()
            buf = bufs[j % 2]

            def body(i, carry, buf=buf):
                s, cnt = carry
                r = i // 8
                cb = (i % 8) * (4 * _LANES)
                for u in range(4):
                    v = buf[r, pl.ds(cb + u * _LANES, _LANES)]
                    msk = v >= tvec
                    s = s + jnp.where(msk, v, 0.0)
                    cnt = cnt + jnp.where(msk, 1, 0)
                return (s, cnt)

            s_acc, c_acc = lax.fori_loop(
                0, chunk_rows * w // (4 * _LANES), body, (s_acc, c_acc))
        osbuf[...] = s_acc
        ocbuf[...] = c_acc
        pltpu.sync_copy(osbuf, sum_out.at[pl.ds(wid * _LANES, _LANES)])
        pltpu.sync_copy(ocbuf, cnt_out.at[pl.ds(wid * _LANES, _LANES)])

    return sc_count_sum


def _count_sum(sc_kernel, loss_parts, t_bits_scalar):
    tf = lax.bitcast_convert_type(jnp.int32(t_bits_scalar), jnp.float32)
    t = jnp.full((_LANES,), tf, dtype=jnp.float32)
    s_tot = jnp.float32(0.0)
    c_tot = jnp.int32(0)
    for part in loss_parts:
        sums, cnts = sc_kernel(part, t)
        s_tot = s_tot + jnp.sum(sums)
        c_tot = c_tot + jnp.sum(cnts)
    return s_tot, c_tot


def kernel(logits, labels):
    b, c, h, w = logits.shape
    n = b * h * w
    n_min = int(n * 0.25)

    nq = 1
    bq = b // nq
    parts = [_ce_loss(logits, labels, 512, q, bq) for q in range(nq)]

    sc_kernel = _make_sc_count_sum(bq * h, w, chunk_rows=32)

    s_hard, n_hard = _count_sum(sc_kernel, parts, _TB_HARD)
    mean_hard = s_hard / n_hard.astype(jnp.float32)

    def topk_mean(_):
        def cond_fn(state):
            lo, hi = state
            return hi - lo > 1

        def body_fn(state):
            lo, hi = state
            mid = lo + (hi - lo) // 2
            _, cnt = _count_sum(sc_kernel, parts, mid)
            ge = cnt >= n_min
            return (jnp.where(ge, mid, lo), jnp.where(ge, hi, mid))

        lo0 = jnp.int32(0)
        hi0 = jnp.int32(0x7F800000)
        v_bits, _ = lax.while_loop(cond_fn, body_fn, (lo0, hi0))
        v = lax.bitcast_convert_type(v_bits, jnp.float32)
        s_gt, c_gt = _count_sum(sc_kernel, parts, v_bits + 1)
        topk_sum = s_gt + (n_min - c_gt).astype(jnp.float32) * v
        return topk_sum / jnp.float32(n_min)

    return lax.cond(n_hard < n_min, topk_mean,
                    lambda _: mean_hard, operand=None)
